# baseline (speedup 1.0000x reference)
import jax
import jax.numpy as jnp
from jax import lax
from jax.experimental import pallas as pl
from jax.experimental.pallas import tpu as pltpu
from jax.experimental.pallas import tpu_sc as plsc

N = 10000
E = 320000
D = 128
CW = 16
DA = D + CW
EPS = 1e-5

NC = 2
NS = 16
K = 96
IB = 14
NB0 = 10
NB1 = 5
C0 = NB0 * IB
C1 = NB1 * IB
TOTC = NS * (C0 + C1)
EPAD = TOTC * K
PAD = EPAD - E
ACC_ROWS = 10112
RPS = ACC_ROWS // NS

_mesh = plsc.VectorSubcoreMesh(
    core_axis_name="c", subcore_axis_name="s", num_cores=NC, num_subcores=NS
)


def _agg_body(nbuf, tbl_hbm, srcw_hbm, dstw_hbm, out_hbm,
              srcb, dstb, *rest):
    rows = rest[:nbuf]
    acc = rest[nbuf]
    gsems = rest[nbuf + 1:2 * nbuf + 1]
    ssems = rest[2 * nbuf + 1:]
    rows0 = rows[0]
    cid = lax.axis_index("c")
    sid = lax.axis_index("s")
    da = rows0.shape[1]
    @pl.loop(0, K)
    def _(r):
        @pl.loop(0, da // 16)
        def _(c):
            rows0[r, pl.ds(c * 16, 16)] = jnp.zeros((16,), jnp.float32)

    base = sid * RPS
    for j in range(6):
        pltpu.sync_copy(rows0, acc.at[pl.ds(base + j * K, K)])
    pltpu.sync_copy(rows0.at[pl.ds(0, RPS - 6 * K)],
                    acc.at[pl.ds(base + 6 * K, RPS - 6 * K)])
    plsc.subcore_barrier()

    is0 = cid == 0
    nb = lax.select(is0, NB0, NB1)
    cbase = lax.select(is0, sid * C0, NS * C0 + sid * C1)

    @pl.loop(0, nb)
    def _(bk):
        off = cbase + bk * IB
        pltpu.sync_copy(srcw_hbm.at[pl.ds(off, IB)], srcb)
        pltpu.sync_copy(dstw_hbm.at[pl.ds(off, IB)], dstb)
        d_g = {}
        d_s = {}
        for j in range(nbuf - 1):
            if j < IB:
                d_g[j] = pltpu.async_copy(
                    tbl_hbm.at[srcb.at[j]], rows[j % nbuf], gsems[j % nbuf])
        for i in range(IB):
            b = i % nbuf
            j = i + nbuf - 1
            if j < IB:
                if i >= 1:
                    d_s[i - 1].wait()
                d_g[j] = pltpu.async_copy(
                    tbl_hbm.at[srcb.at[j]], rows[j % nbuf], gsems[j % nbuf])
            d_g[i].wait()
            d_s[i] = pltpu.async_copy(rows[b], acc.at[dstb.at[i]],
                                      ssems[b], add=True)
        for i in range(max(0, IB - nbuf), IB):
            d_s[i].wait()

    plsc.subcore_barrier()
    pltpu.sync_copy(acc.at[pl.ds(base, RPS)], out_hbm.at[cid].at[pl.ds(base, RPS)])


def _make_agg(da, nbuf):
    import functools
    return pl.kernel(
        functools.partial(_agg_body, nbuf),
        out_type=jax.ShapeDtypeStruct((NC, ACC_ROWS, da), jnp.float32),
        mesh=_mesh,
        scratch_types=(
            [pltpu.VMEM((IB, K), jnp.int32),
             pltpu.VMEM((IB, K), jnp.int32)]
            + [pltpu.VMEM((K, da), jnp.float32)] * nbuf
            + [pltpu.VMEM_SHARED((ACC_ROWS, da), jnp.float32)]
            + [pltpu.SemaphoreType.DMA] * (2 * nbuf)
        ),
        compiler_params=pltpu.CompilerParams(use_tc_tiling_on_sc=False),
    )


_agg_aug = _make_agg(DA, 2)
_agg_plain = _make_agg(D, 3)


def _lin_body(x_ref, w_ref, o_ref):
    o_ref[...] = lax.dot_general(
        x_ref[...], w_ref[...], (((1,), (1,)), ((), ())),
        preferred_element_type=jnp.float32)


def _linear(x, w):
    return pl.pallas_call(
        _lin_body,
        out_shape=jax.ShapeDtypeStruct((x.shape[0], w.shape[0]), jnp.float32),
    )(x, w)


def _sage_tail(aggr, cnt, xr, wl, bl, g, b):
    mean = aggr / jnp.maximum(cnt, 1.0)
    pre = lax.dot_general(
        mean, wl, (((1,), (1,)), ((), ())),
        preferred_element_type=jnp.float32)
    pre = pre + bl + xr
    mu = jnp.mean(pre, axis=0, keepdims=True)
    var = jnp.mean((pre - mu) ** 2, axis=0, keepdims=True)
    hn = (pre - mu) * lax.rsqrt(var + EPS) * g + b
    return jnp.maximum(hn, 0.0)


def _dense1_body(s_ref, xr_ref, wl_ref, bl_ref, g_ref, b_ref, o_ref):
    s = s_ref[...]
    aggr = s[0, :N, :D] + s[1, :N, :D]
    cnt = s[0, :N, D:D + 1] + s[1, :N, D:D + 1]
    o_ref[...] = _sage_tail(aggr, cnt, xr_ref[...], wl_ref[...],
                            bl_ref[...], g_ref[...], b_ref[...])


def _dense1(sums, xr, wl, bl, g, b):
    return pl.pallas_call(
        _dense1_body,
        out_shape=jax.ShapeDtypeStruct((N, D), jnp.float32),
    )(sums, xr, wl.reshape(D, D), bl.reshape(1, D),
      g.reshape(1, D), b.reshape(1, D))


def _dense2_body(s_ref, c_ref, xr_ref, wl_ref, bl_ref, g_ref, b_ref,
                 wlin_ref, blin_ref, o_ref):
    s = s_ref[...]
    c = c_ref[...]
    aggr = s[0, :N, :] + s[1, :N, :]
    cnt = c[0, :N, 0:1] + c[1, :N, 0:1]
    h = _sage_tail(aggr, cnt, xr_ref[...], wl_ref[...],
                   bl_ref[...], g_ref[...], b_ref[...])
    o_ref[...] = lax.dot_general(
        h, wlin_ref[...], (((1,), (1,)), ((), ())),
        preferred_element_type=jnp.float32) + blin_ref[...]


def _dense2(sums, cnts, xr, wl, bl, g, b, wlin, blin):
    return pl.pallas_call(
        _dense2_body,
        out_shape=jax.ShapeDtypeStruct((N, wlin.shape[0]), jnp.float32),
    )(sums, cnts, xr, wl.reshape(D, D), bl.reshape(1, D),
      g.reshape(1, D), b.reshape(1, D), wlin, blin.reshape(1, -1))


def kernel(x, edge_index, Wl1, bl1, Wr1, g1, b1, Wl2, bl2, Wr2, g2, b2, Wlin, blin):
    src = edge_index[0]
    dst = edge_index[1]
    srcw = jnp.concatenate([src, jnp.zeros((PAD,), jnp.int32)]).reshape(TOTC, K)
    dstw = jnp.concatenate([dst, jnp.full((PAD,), N, jnp.int32)]).reshape(TOTC, K)
    x_aug = jnp.concatenate([x, jnp.ones((N, CW), jnp.float32)], axis=1)

    sums1 = _agg_aug(x_aug, srcw, dstw)
    xr1 = _linear(x, Wr1)
    h = _dense1(sums1, xr1, Wl1, bl1, g1, b1)

    sums2 = _agg_plain(h, srcw, dstw)
    hr2 = _linear(h, Wr2)
    cnts = sums1[:, :, D:]
    return _dense2(sums2, cnts, hr2, Wl2, bl2, g2, b2, Wlin, blin)


# speedup vs baseline: 1.0168x; 1.0168x over previous
import jax
import jax.numpy as jnp
from jax import lax
from jax.experimental import pallas as pl
from jax.experimental.pallas import tpu as pltpu
from jax.experimental.pallas import tpu_sc as plsc

N = 10000
E = 320000
D = 128
CW = 16
DA = D + CW
EPS = 1e-5

NC = 2
NS = 16
K = 96
IB = 14
NB0 = 10
NB1 = 5
C0 = NB0 * IB
C1 = NB1 * IB
TOTC = NS * (C0 + C1)
EPAD = TOTC * K
PAD = EPAD - E
ACC_ROWS = 10112
RPS = ACC_ROWS // NS

_mesh = plsc.VectorSubcoreMesh(
    core_axis_name="c", subcore_axis_name="s", num_cores=NC, num_subcores=NS
)


def _agg_body(nbuf, tbl_hbm, srcw_hbm, dstw_hbm, out_hbm,
              srcb, dstb, *rest):
    rows = rest[:nbuf]
    acc = rest[nbuf]
    gsems = rest[nbuf + 1:2 * nbuf + 1]
    ssems = rest[2 * nbuf + 1:]
    rows0 = rows[0]
    cid = lax.axis_index("c")
    sid = lax.axis_index("s")
    da = rows0.shape[1]
    @pl.loop(0, K)
    def _(r):
        @pl.loop(0, da // 16)
        def _(c):
            rows0[r, pl.ds(c * 16, 16)] = jnp.zeros((16,), jnp.float32)

    base = sid * RPS
    for j in range(6):
        pltpu.sync_copy(rows0, acc.at[pl.ds(base + j * K, K)])
    pltpu.sync_copy(rows0.at[pl.ds(0, RPS - 6 * K)],
                    acc.at[pl.ds(base + 6 * K, RPS - 6 * K)])
    plsc.subcore_barrier()

    is0 = cid == 0
    nb = lax.select(is0, NB0, NB1)
    cbase = lax.select(is0, sid * C0, NS * C0 + sid * C1)

    @pl.loop(0, nb)
    def _(bk):
        off = cbase + bk * IB
        pltpu.sync_copy(srcw_hbm.at[pl.ds(off, IB)], srcb)
        pltpu.sync_copy(dstw_hbm.at[pl.ds(off, IB)], dstb)
        d_g = {}
        d_s = {}
        for j in range(nbuf - 1):
            if j < IB:
                d_g[j] = pltpu.async_copy(
                    tbl_hbm.at[srcb.at[j]], rows[j % nbuf], gsems[j % nbuf])
        for i in range(IB):
            b = i % nbuf
            j = i + nbuf - 1
            if j < IB:
                if i >= 1:
                    d_s[i - 1].wait()
                d_g[j] = pltpu.async_copy(
                    tbl_hbm.at[srcb.at[j]], rows[j % nbuf], gsems[j % nbuf])
            d_g[i].wait()
            d_s[i] = pltpu.async_copy(rows[b], acc.at[dstb.at[i]],
                                      ssems[b], add=True)
        for i in range(max(0, IB - nbuf), IB):
            d_s[i].wait()

    plsc.subcore_barrier()
    pltpu.sync_copy(acc.at[pl.ds(base, RPS)], out_hbm.at[cid].at[pl.ds(base, RPS)])


def _make_agg(da, nbuf):
    import functools
    return pl.kernel(
        functools.partial(_agg_body, nbuf),
        out_type=jax.ShapeDtypeStruct((NC, ACC_ROWS, da), jnp.float32),
        mesh=_mesh,
        scratch_types=(
            [pltpu.VMEM((IB, K), jnp.int32),
             pltpu.VMEM((IB, K), jnp.int32)]
            + [pltpu.VMEM((K, da), jnp.float32)] * nbuf
            + [pltpu.VMEM_SHARED((ACC_ROWS, da), jnp.float32)]
            + [pltpu.SemaphoreType.DMA] * (2 * nbuf)
        ),
        compiler_params=pltpu.CompilerParams(use_tc_tiling_on_sc=False),
    )


_agg_aug = _make_agg(DA, 2)
_agg_plain = _make_agg(D, 3)


def _lin_body(x_ref, w_ref, o_ref):
    o_ref[...] = lax.dot_general(
        x_ref[...], w_ref[...], (((1,), (1,)), ((), ())),
        preferred_element_type=jnp.float32)


def _linear(x, w):
    return pl.pallas_call(
        _lin_body,
        out_shape=jax.ShapeDtypeStruct((x.shape[0], w.shape[0]), jnp.float32),
    )(x, w)


def _sage_tail(aggr, cnt, xr, wl, bl, g, b):
    mean = aggr / jnp.maximum(cnt, 1.0)
    pre = lax.dot_general(
        mean, wl, (((1,), (1,)), ((), ())),
        preferred_element_type=jnp.float32)
    pre = pre + bl + xr
    mu = jnp.mean(pre, axis=0, keepdims=True)
    var = jnp.mean((pre - mu) ** 2, axis=0, keepdims=True)
    hn = (pre - mu) * lax.rsqrt(var + EPS) * g + b
    return jnp.maximum(hn, 0.0)


def _dense1_body(s_ref, xr_ref, wl_ref, bl_ref, g_ref, b_ref, o_ref):
    s = s_ref[...]
    aggr = s[0, :N, :D] + s[1, :N, :D]
    cnt = s[0, :N, D:D + 1] + s[1, :N, D:D + 1]
    o_ref[...] = _sage_tail(aggr, cnt, xr_ref[...], wl_ref[...],
                            bl_ref[...], g_ref[...], b_ref[...])


def _dense1(sums, xr, wl, bl, g, b):
    return pl.pallas_call(
        _dense1_body,
        out_shape=jax.ShapeDtypeStruct((N, D), jnp.float32),
    )(sums, xr, wl.reshape(D, D), bl.reshape(1, D),
      g.reshape(1, D), b.reshape(1, D))


def _dense2_body(s_ref, c_ref, xr_ref, wl_ref, bl_ref, g_ref, b_ref,
                 wlin_ref, blin_ref, o_ref):
    s = s_ref[...]
    c = c_ref[...]
    aggr = s[0, :N, :] + s[1, :N, :]
    cnt = c[0, :N, 0:1] + c[1, :N, 0:1]
    h = _sage_tail(aggr, cnt, xr_ref[...], wl_ref[...],
                   bl_ref[...], g_ref[...], b_ref[...])
    o_ref[...] = lax.dot_general(
        h, wlin_ref[...], (((1,), (1,)), ((), ())),
        preferred_element_type=jnp.float32) + blin_ref[...]


def _dense2(sums, cnts, xr, wl, bl, g, b, wlin, blin):
    return pl.pallas_call(
        _dense2_body,
        out_shape=jax.ShapeDtypeStruct((N, wlin.shape[0]), jnp.float32),
    )(sums, cnts, xr, wl.reshape(D, D), bl.reshape(1, D),
      g.reshape(1, D), b.reshape(1, D), wlin, blin.reshape(1, -1))


def kernel(x, edge_index, Wl1, bl1, Wr1, g1, b1, Wl2, bl2, Wr2, g2, b2, Wlin, blin):
    src = edge_index[0]
    dst = edge_index[1]
    srcw = jnp.concatenate([src, jnp.zeros((PAD,), jnp.int32)]).reshape(TOTC, K)
    junk = N + jnp.arange(PAD, dtype=jnp.int32) % (ACC_ROWS - N)
    dstw = jnp.concatenate([dst, junk]).reshape(TOTC, K)
    x_aug = jnp.concatenate([x, jnp.ones((N, CW), jnp.float32)], axis=1)

    sums1 = _agg_aug(x_aug, srcw, dstw)
    xr1 = _linear(x, Wr1)
    h = _dense1(sums1, xr1, Wl1, bl1, g1, b1)

    sums2 = _agg_plain(h, srcw, dstw)
    hr2 = _linear(h, Wr2)
    cnts = sums1[:, :, D:]
    return _dense2(sums2, cnts, hr2, Wl2, bl2, g2, b2, Wlin, blin)
